# baseline (speedup 1.0000x reference)
import functools

import jax
import jax.numpy as jnp
from jax import lax
from jax.experimental import pallas as pl
from jax.experimental.pallas import tpu as pltpu
from jax.experimental.pallas import tpu_sc as plsc

_BATCH = 16384
_D = 64
_W = 2 * _D
_NW = 32
_CHUNK = 128
_ROWS_PER_W = _BATCH // _NW
_HALF = 256
_CPH = _HALF // _CHUNK


def _sc_gather_body(U_hbm, M_hbm, uidx_hbm, midx_hbm, ue_hbm, me_hbm,
                    uidx_v, midx_v, urows_v, mrows_v, sem_u, sem_m):
    wid = lax.axis_index("s") * 2 + lax.axis_index("c")
    for h in range(2):
        row0 = wid * 2 * _CPH + h * _CPH
        pltpu.sync_copy(uidx_hbm.at[pl.ds(row0, _CPH)], uidx_v)
        pltpu.sync_copy(midx_hbm.at[pl.ds(row0, _CPH)], midx_v)
        copies = []
        for j in range(_CPH):
            dst = urows_v.at[pl.ds(j * _CHUNK, _CHUNK)]
            copies.append(pltpu.async_copy(U_hbm.at[uidx_v.at[j]], dst, sem_u))
            dst = mrows_v.at[pl.ds(j * _CHUNK, _CHUNK)]
            copies.append(pltpu.async_copy(M_hbm.at[midx_v.at[j]], dst, sem_m))
        for c in copies:
            c.wait()
        base = wid * _ROWS_PER_W + h * _HALF
        pltpu.sync_copy(urows_v, ue_hbm.at[pl.ds(base, _HALF)])
        pltpu.sync_copy(mrows_v, me_hbm.at[pl.ds(base, _HALF)])


def _sc_gather(U2, M2, users2d, movies2d):
    mesh = plsc.VectorSubcoreMesh(core_axis_name="c", subcore_axis_name="s")
    k = functools.partial(
        pl.kernel,
        mesh=mesh,
        out_type=[
            jax.ShapeDtypeStruct((_BATCH, _W), jnp.float32),
            jax.ShapeDtypeStruct((_BATCH, _W), jnp.float32),
        ],
        scratch_types=[
            pltpu.VMEM((_CPH, _CHUNK), jnp.int32),
            pltpu.VMEM((_CPH, _CHUNK), jnp.int32),
            pltpu.VMEM((_HALF, _W), jnp.float32),
            pltpu.VMEM((_HALF, _W), jnp.float32),
            pltpu.SemaphoreType.DMA,
            pltpu.SemaphoreType.DMA,
        ],
    )(_sc_gather_body)
    return k(U2, M2, users2d, movies2d)


def _mlp_body(ue_ref, me_ref, g_ref, up_ref, mp_ref, w1_ref, b1_ref,
              w2_ref, b2_ref, wf_ref, bf_ref, out_ref):
    ue = jnp.where(up_ref[...] > 0, ue_ref[:, _D:], ue_ref[:, :_D])
    me = jnp.where(mp_ref[...] > 0, me_ref[:, _D:], me_ref[:, :_D])
    w1 = w1_ref[...]
    h = ue @ w1[0:_D, :]
    h += me @ w1[_D:2 * _D, :]
    h += g_ref[...] @ w1[2 * _D:, :]
    h = jnp.maximum(h + b1_ref[...], 0.0)
    h = jnp.maximum(h @ w2_ref[...] + b2_ref[...], 0.0)
    o = jnp.sum(h * wf_ref[...], axis=1, keepdims=True) + bf_ref[...]
    out_ref[...] = 1.0 / (1.0 + jnp.exp(-o))


def _mlp(ue, me, genres, uparity, mparity, W1, b1, W2, b2, Wf, bf):
    bb = 2048
    grid = (_BATCH // bb,)
    n_in = 2 * _D + genres.shape[1]
    h1, h2 = W1.shape[1], W2.shape[1]
    return pl.pallas_call(
        _mlp_body,
        grid=grid,
        in_specs=[
            pl.BlockSpec((bb, _W), lambda i: (i, 0)),
            pl.BlockSpec((bb, _W), lambda i: (i, 0)),
            pl.BlockSpec((bb, genres.shape[1]), lambda i: (i, 0)),
            pl.BlockSpec((bb, 1), lambda i: (i, 0)),
            pl.BlockSpec((bb, 1), lambda i: (i, 0)),
            pl.BlockSpec((n_in, h1), lambda i: (0, 0)),
            pl.BlockSpec((1, h1), lambda i: (0, 0)),
            pl.BlockSpec((h1, h2), lambda i: (0, 0)),
            pl.BlockSpec((1, h2), lambda i: (0, 0)),
            pl.BlockSpec((1, h2), lambda i: (0, 0)),
            pl.BlockSpec((1, 1), lambda i: (0, 0)),
        ],
        out_specs=pl.BlockSpec((bb, 1), lambda i: (i, 0)),
        out_shape=jax.ShapeDtypeStruct((_BATCH, 1), jnp.float32),
    )(ue, me, genres, uparity, mparity, W1, b1, W2, b2, Wf, bf)


def kernel(users, movies, genres, U, M, W1, b1, W2, b2, Wf, bf):
    users = users.astype(jnp.int32)
    movies = movies.astype(jnp.int32)
    U2 = U.reshape(-1, _W)
    M2 = M.reshape(-1, _W)
    users2d = (users // 2).reshape(_BATCH // _CHUNK, _CHUNK)
    movies2d = (movies // 2).reshape(_BATCH // _CHUNK, _CHUNK)
    uparity = (users % 2).reshape(_BATCH, 1)
    mparity = (movies % 2).reshape(_BATCH, 1)
    ue, me = _sc_gather(U2, M2, users2d, movies2d)
    return _mlp(ue, me, genres, uparity, mparity,
                W1, b1.reshape(1, -1), W2, b2.reshape(1, -1),
                Wf.reshape(1, -1), bf.reshape(1, 1))


# speedup vs baseline: 1.9287x; 1.9287x over previous
import functools

import jax
import jax.numpy as jnp
from jax import lax
from jax.experimental import pallas as pl
from jax.experimental.pallas import tpu as pltpu
from jax.experimental.pallas import tpu_sc as plsc

_BATCH = 16384
_D = 64
_W = 2 * _D
_NW = 32
_CHUNK = 128
_ROWS_PER_W = _BATCH // _NW
_HALF = 256
_CPH = _HALF // _CHUNK
_US = 512000
_UTB = 4096
_MS = 51200
_MTB = 2048


def _transpose_body(a_ref, b_ref, out_ref):
    eye = (lax.broadcasted_iota(jnp.int32, (_D, _D), 0)
           == lax.broadcasted_iota(jnp.int32, (_D, _D), 1)).astype(jnp.float32)
    dn = (((0,), (0,)), ((), ()))
    at = lax.dot_general(a_ref[...], eye, dn,
                         preferred_element_type=jnp.float32)
    bt = lax.dot_general(b_ref[...], eye, dn,
                         preferred_element_type=jnp.float32)
    out_ref[...] = jnp.concatenate([at, bt], axis=1)


def _pack_rows(XT, s_rows, tb):
    nb = s_rows // tb
    last = (XT.shape[1] + tb - 1) // tb - 1
    return pl.pallas_call(
        _transpose_body,
        grid=(nb,),
        in_specs=[
            pl.BlockSpec((_D, tb), lambda i: (0, i)),
            pl.BlockSpec((_D, tb), lambda i: (0, jnp.minimum(i + nb, last))),
        ],
        out_specs=pl.BlockSpec((tb, _W), lambda i: (i, 0)),
        out_shape=jax.ShapeDtypeStruct((s_rows, _W), jnp.float32),
    )(XT, XT)


def _sc_gather_body(U_hbm, M_hbm, uidx_hbm, midx_hbm, ue_hbm, me_hbm,
                    uidx_v, midx_v, urows_v, mrows_v, sem_u, sem_m):
    wid = lax.axis_index("s") * 2 + lax.axis_index("c")
    for h in range(2):
        row0 = wid * 2 * _CPH + h * _CPH
        pltpu.sync_copy(uidx_hbm.at[pl.ds(row0, _CPH)], uidx_v)
        pltpu.sync_copy(midx_hbm.at[pl.ds(row0, _CPH)], midx_v)
        copies = []
        for j in range(_CPH):
            dst = urows_v.at[pl.ds(j * _CHUNK, _CHUNK)]
            copies.append(pltpu.async_copy(U_hbm.at[uidx_v.at[j]], dst, sem_u))
            dst = mrows_v.at[pl.ds(j * _CHUNK, _CHUNK)]
            copies.append(pltpu.async_copy(M_hbm.at[midx_v.at[j]], dst, sem_m))
        for c in copies:
            c.wait()
        base = wid * _ROWS_PER_W + h * _HALF
        pltpu.sync_copy(urows_v, ue_hbm.at[pl.ds(base, _HALF)])
        pltpu.sync_copy(mrows_v, me_hbm.at[pl.ds(base, _HALF)])


def _sc_gather(U2, M2, users2d, movies2d):
    mesh = plsc.VectorSubcoreMesh(core_axis_name="c", subcore_axis_name="s")
    k = functools.partial(
        pl.kernel,
        mesh=mesh,
        out_type=[
            jax.ShapeDtypeStruct((_BATCH, _W), jnp.float32),
            jax.ShapeDtypeStruct((_BATCH, _W), jnp.float32),
        ],
        scratch_types=[
            pltpu.VMEM((_CPH, _CHUNK), jnp.int32),
            pltpu.VMEM((_CPH, _CHUNK), jnp.int32),
            pltpu.VMEM((_HALF, _W), jnp.float32),
            pltpu.VMEM((_HALF, _W), jnp.float32),
            pltpu.SemaphoreType.DMA,
            pltpu.SemaphoreType.DMA,
        ],
    )(_sc_gather_body)
    return k(U2, M2, users2d, movies2d)


def _mlp_body(ue_ref, me_ref, g_ref, up_ref, mp_ref, w1_ref, b1_ref,
              w2_ref, b2_ref, wf_ref, bf_ref, out_ref):
    ue = jnp.where(up_ref[...] > 0, ue_ref[:, _D:], ue_ref[:, :_D])
    me = jnp.where(mp_ref[...] > 0, me_ref[:, _D:], me_ref[:, :_D])
    w1 = w1_ref[...]
    h = ue @ w1[0:_D, :]
    h += me @ w1[_D:2 * _D, :]
    h += g_ref[...] @ w1[2 * _D:, :]
    h = jnp.maximum(h + b1_ref[...], 0.0)
    h = jnp.maximum(h @ w2_ref[...] + b2_ref[...], 0.0)
    o = jnp.sum(h * wf_ref[...], axis=1, keepdims=True) + bf_ref[...]
    out_ref[...] = 1.0 / (1.0 + jnp.exp(-o))


def _mlp(ue, me, genres, uparity, mparity, W1, b1, W2, b2, Wf, bf):
    bb = 2048
    grid = (_BATCH // bb,)
    n_in = 2 * _D + genres.shape[1]
    h1, h2 = W1.shape[1], W2.shape[1]
    return pl.pallas_call(
        _mlp_body,
        grid=grid,
        in_specs=[
            pl.BlockSpec((bb, _W), lambda i: (i, 0)),
            pl.BlockSpec((bb, _W), lambda i: (i, 0)),
            pl.BlockSpec((bb, genres.shape[1]), lambda i: (i, 0)),
            pl.BlockSpec((bb, 1), lambda i: (i, 0)),
            pl.BlockSpec((bb, 1), lambda i: (i, 0)),
            pl.BlockSpec((n_in, h1), lambda i: (0, 0)),
            pl.BlockSpec((1, h1), lambda i: (0, 0)),
            pl.BlockSpec((h1, h2), lambda i: (0, 0)),
            pl.BlockSpec((1, h2), lambda i: (0, 0)),
            pl.BlockSpec((1, h2), lambda i: (0, 0)),
            pl.BlockSpec((1, 1), lambda i: (0, 0)),
        ],
        out_specs=pl.BlockSpec((bb, 1), lambda i: (i, 0)),
        out_shape=jax.ShapeDtypeStruct((_BATCH, 1), jnp.float32),
    )(ue, me, genres, uparity, mparity, W1, b1, W2, b2, Wf, bf)


def kernel(users, movies, genres, U, M, W1, b1, W2, b2, Wf, bf):
    users = users.astype(jnp.int32)
    movies = movies.astype(jnp.int32)
    U2 = _pack_rows(U.T, _US, _UTB)
    M2 = _pack_rows(M.T, _MS, _MTB)
    users2d = (users % _US).reshape(_BATCH // _CHUNK, _CHUNK)
    movies2d = (movies % _MS).reshape(_BATCH // _CHUNK, _CHUNK)
    uparity = (users // _US).reshape(_BATCH, 1)
    mparity = (movies // _MS).reshape(_BATCH, 1)
    ue, me = _sc_gather(U2, M2, users2d, movies2d)
    return _mlp(ue, me, genres, uparity, mparity,
                W1, b1.reshape(1, -1), W2, b2.reshape(1, -1),
                Wf.reshape(1, -1), bf.reshape(1, 1))


# speedup vs baseline: 2.1152x; 1.0967x over previous
import functools

import jax
import jax.numpy as jnp
from jax import lax
from jax.experimental import pallas as pl
from jax.experimental.pallas import tpu as pltpu
from jax.experimental.pallas import tpu_sc as plsc

_BATCH = 16384
_D = 64
_W = 2 * _D
_NW = 32
_CHUNK = 128
_ROWS_PER_W = _BATCH // _NW
_HALF = 256
_CPH = _HALF // _CHUNK
_US = 512000
_MS = 51200
_TB = 6400


def _transpose_body(a_ref, b_ref, out_ref):
    eye = (lax.broadcasted_iota(jnp.int32, (_D, _D), 0)
           == lax.broadcasted_iota(jnp.int32, (_D, _D), 1)).astype(jnp.float32)
    dn = (((0,), (0,)), ((), ()))
    at = lax.dot_general(a_ref[...], eye, dn,
                         preferred_element_type=jnp.float32)
    bt = lax.dot_general(b_ref[...], eye, dn,
                         preferred_element_type=jnp.float32)
    out_ref[...] = jnp.concatenate([at, bt], axis=1)


def _pack_rows(XT, s_rows):
    nb = s_rows // _TB
    last = (XT.shape[1] + _TB - 1) // _TB - 1
    return pl.pallas_call(
        _transpose_body,
        grid=(nb,),
        in_specs=[
            pl.BlockSpec((_D, _TB), lambda i: (0, i)),
            pl.BlockSpec((_D, _TB), lambda i: (0, jnp.minimum(i + nb, last))),
        ],
        out_specs=pl.BlockSpec((_TB, _W), lambda i: (i, 0)),
        out_shape=jax.ShapeDtypeStruct((s_rows, _W), jnp.float32),
    )(XT, XT)


def _sc_gather_body(T_hbm, idx_hbm, out_hbm, idx_v, rows_v, sem):
    wid = lax.axis_index("s") * 2 + lax.axis_index("c")
    for h in range(2):
        row0 = wid * 2 * _CPH + h * _CPH
        pltpu.sync_copy(idx_hbm.at[pl.ds(row0, _CPH)], idx_v)
        copies = []
        for j in range(_CPH):
            dst = rows_v.at[pl.ds(j * _CHUNK, _CHUNK)]
            copies.append(pltpu.async_copy(T_hbm.at[idx_v.at[j]], dst, sem))
        for c in copies:
            c.wait()
        base = wid * _ROWS_PER_W + h * _HALF
        pltpu.sync_copy(rows_v, out_hbm.at[pl.ds(base, _HALF)])


def _sc_gather(T2, idx2d):
    mesh = plsc.VectorSubcoreMesh(core_axis_name="c", subcore_axis_name="s")
    k = functools.partial(
        pl.kernel,
        mesh=mesh,
        out_type=jax.ShapeDtypeStruct((_BATCH, _W), jnp.float32),
        scratch_types=[
            pltpu.VMEM((_CPH, _CHUNK), jnp.int32),
            pltpu.VMEM((_HALF, _W), jnp.float32),
            pltpu.SemaphoreType.DMA,
        ],
    )(_sc_gather_body)
    return k(T2, idx2d)


def _mlp_body(ue_ref, me_ref, g_ref, up_ref, mp_ref, w1_ref, b1_ref,
              w2_ref, b2_ref, wf_ref, bf_ref, out_ref):
    ue = jnp.where(up_ref[...] > 0, ue_ref[:, _D:], ue_ref[:, :_D])
    me = jnp.where(mp_ref[...] > 0, me_ref[:, _D:], me_ref[:, :_D])
    w1 = w1_ref[...]
    h = ue @ w1[0:_D, :]
    h += me @ w1[_D:2 * _D, :]
    h += g_ref[...] @ w1[2 * _D:, :]
    h = jnp.maximum(h + b1_ref[...], 0.0)
    h = jnp.maximum(h @ w2_ref[...] + b2_ref[...], 0.0)
    o = jnp.sum(h * wf_ref[...], axis=1, keepdims=True) + bf_ref[...]
    out_ref[...] = 1.0 / (1.0 + jnp.exp(-o))


def _mlp(ue, me, genres, uparity, mparity, W1, b1, W2, b2, Wf, bf):
    bb = 2048
    grid = (_BATCH // bb,)
    n_in = 2 * _D + genres.shape[1]
    h1, h2 = W1.shape[1], W2.shape[1]
    return pl.pallas_call(
        _mlp_body,
        grid=grid,
        in_specs=[
            pl.BlockSpec((bb, _W), lambda i: (i, 0)),
            pl.BlockSpec((bb, _W), lambda i: (i, 0)),
            pl.BlockSpec((bb, genres.shape[1]), lambda i: (i, 0)),
            pl.BlockSpec((bb, 1), lambda i: (i, 0)),
            pl.BlockSpec((bb, 1), lambda i: (i, 0)),
            pl.BlockSpec((n_in, h1), lambda i: (0, 0)),
            pl.BlockSpec((1, h1), lambda i: (0, 0)),
            pl.BlockSpec((h1, h2), lambda i: (0, 0)),
            pl.BlockSpec((1, h2), lambda i: (0, 0)),
            pl.BlockSpec((1, h2), lambda i: (0, 0)),
            pl.BlockSpec((1, 1), lambda i: (0, 0)),
        ],
        out_specs=pl.BlockSpec((bb, 1), lambda i: (i, 0)),
        out_shape=jax.ShapeDtypeStruct((_BATCH, 1), jnp.float32),
    )(ue, me, genres, uparity, mparity, W1, b1, W2, b2, Wf, bf)


def kernel(users, movies, genres, U, M, W1, b1, W2, b2, Wf, bf):
    users = users.astype(jnp.int32)
    movies = movies.astype(jnp.int32)
    U2 = _pack_rows(U.T, _US)
    M2 = _pack_rows(M.T, _MS)
    users2d = (users % _US).reshape(_BATCH // _CHUNK, _CHUNK)
    movies2d = (movies % _MS).reshape(_BATCH // _CHUNK, _CHUNK)
    uparity = (users // _US).reshape(_BATCH, 1)
    mparity = (movies // _MS).reshape(_BATCH, 1)
    ue = _sc_gather(U2, users2d)
    me = _sc_gather(M2, movies2d)
    return _mlp(ue, me, genres, uparity, mparity,
                W1, b1.reshape(1, -1), W2, b2.reshape(1, -1),
                Wf.reshape(1, -1), bf.reshape(1, 1))
